# baseline (speedup 1.0000x reference)
import functools

import jax
import jax.numpy as jnp
import numpy as np
from jax import lax
from jax.experimental import pallas as pl
from jax.experimental.pallas import tpu as pltpu
from jax.experimental.pallas import tpu_sc as plsc

D_MODEL = 256
N_HEADS = 8
N_LEVELS = 4
N_POINTS = 4
HEAD_DIM = 32
SPATIAL_SHAPES = [(64, 64), (32, 32), (16, 16), (8, 8)]
NV = sum(h * w for h, w in SPATIAL_SHAPES)

_LVL = (np.arange(128) // 4) % 4
_WF = np.array([SPATIAL_SHAPES[l][1] for l in _LVL], np.float32).reshape(1, 128)
_HF = np.array([SPATIAL_SHAPES[l][0] for l in _LVL], np.float32).reshape(1, 128)
_WI = _WF.astype(np.int32)
_STARTS = np.cumsum([0] + [h * w for h, w in SPATIAL_SHAPES])[:4]
_STARTI = np.array([_STARTS[l] for l in _LVL], np.int32).reshape(1, 128)
_HCOL = (np.arange(128) // 16).astype(np.int32).reshape(1, 128)
_SX = np.zeros((8, 128), np.float32)
_SY = np.zeros((8, 128), np.float32)
for _l in range(4):
    _SX[2 * _l] = (_LVL == _l)
    _SY[2 * _l + 1] = (_LVL == _l)
_S16 = ((np.arange(128)[:, None] // 16) == (np.arange(128)[None, :] // 16)
        ).astype(np.float32)
_UNPACK_PERM = np.concatenate(
    [h * 32 + np.concatenate([np.arange(16) * 2, np.arange(16) * 2 + 1])
     for h in range(N_HEADS)])


def _matmul_body(out_dtype, x_ref, w_ref, b_ref, o_ref):
    o_ref[...] = (
        jnp.dot(x_ref[...], w_ref[...], preferred_element_type=jnp.float32)
        + b_ref[...]
    ).astype(out_dtype)


def _matmul(x, w, b, block_rows, out_dtype=jnp.float32):
    n, k = x.shape
    m = w.shape[1]
    grid = n // block_rows
    return pl.pallas_call(
        functools.partial(_matmul_body, out_dtype),
        grid=(grid,),
        in_specs=[
            pl.BlockSpec((block_rows, k), lambda i: (i, 0)),
            pl.BlockSpec((k, m), lambda i: (0, 0)),
            pl.BlockSpec((1, m), lambda i: (0, 0)),
        ],
        out_specs=pl.BlockSpec((block_rows, m), lambda i: (i, 0)),
        out_shape=jax.ShapeDtypeStruct((n, m), out_dtype),
    )(x, w, b.reshape(1, m))


_ROWS_PER_STEP = 1800


def _sampler_body(q_ref, rp_ref, wh_ref, wl_ref, bx_ref, by_ref,
                  wa_ref, ba_ref, sx_ref, sy_ref, s16_ref, wf_ref, hf_ref,
                  wi_ref, starti_ref, hcol_ref, iw_ref):
    step = pl.program_id(0)
    r = _ROWS_PER_STEP
    q = q_ref[...]
    qh = q.astype(jnp.bfloat16)
    ql = (q - qh.astype(jnp.float32)).astype(jnp.bfloat16)
    wh = wh_ref[...]
    oxy = (jnp.dot(qh, wh, preferred_element_type=jnp.float32)
           + jnp.dot(qh, wl_ref[...], preferred_element_type=jnp.float32)
           + jnp.dot(ql, wh, preferred_element_type=jnp.float32))
    offx = oxy[:, 0:128] + bx_ref[...]
    offy = oxy[:, 128:256] + by_ref[...]
    a = jnp.dot(q, wa_ref[...], preferred_element_type=jnp.float32) + ba_ref[...]
    e = jnp.exp(a - jnp.max(a, axis=1, keepdims=True))
    attn = e / jnp.dot(e, s16_ref[...], preferred_element_type=jnp.float32)
    rpf = rp_ref[...]
    rph = rpf.astype(jnp.bfloat16)
    rpl = (rpf - rph.astype(jnp.float32)).astype(jnp.bfloat16)
    sx = sx_ref[...].astype(jnp.bfloat16)
    sy = sy_ref[...].astype(jnp.bfloat16)
    rx = (jnp.dot(rph, sx, preferred_element_type=jnp.float32)
          + jnp.dot(rpl, sx, preferred_element_type=jnp.float32))
    ry = (jnp.dot(rph, sy, preferred_element_type=jnp.float32)
          + jnp.dot(rpl, sy, preferred_element_type=jnp.float32))
    wf = wf_ref[...]
    hf = hf_ref[...]
    x = rx * wf + offx - 0.5
    y = ry * hf + offy - 0.5
    x0 = jnp.floor(x)
    fx = x - x0
    y0 = jnp.floor(y)
    fy = y - y0
    rg = step * r + lax.broadcasted_iota(jnp.int32, (r, 128), 0)
    bvec = ((rg >= 900).astype(jnp.int32) + (rg >= 1800).astype(jnp.int32)
            + (rg >= 2700).astype(jnp.int32))
    base = bvec * (NV * N_HEADS)
    for cy in (0, 1):
        yf = y0 + cy
        vy = (yf >= 0.0) & (yf <= hf - 1.0)
        yc = jnp.clip(yf, 0.0, hf - 1.0).astype(jnp.int32)
        wyv = fy if cy else 1.0 - fy
        for cx in (0, 1):
            xf = x0 + cx
            vx = (xf >= 0.0) & (xf <= wf - 1.0)
            xc = jnp.clip(xf, 0.0, wf - 1.0).astype(jnp.int32)
            wxv = fx if cx else 1.0 - fx
            wgt = jnp.where(vx & vy, attn * wxv * wyv, 0.0)
            pix = starti_ref[...] + yc * wi_ref[...] + xc
            idx = base + pix * N_HEADS + hcol_ref[...]
            c = cy * 2 + cx
            iw_ref[:, 0, c, :] = idx
            iw_ref[:, 1, c, :] = (wgt * (2.0 ** 20)).astype(jnp.int32)


def _sampler(qflat, rp8, wh, wl, bx, by, wa, ba):
    n = qflat.shape[0]
    grid = n // _ROWS_PER_STEP
    r = _ROWS_PER_STEP
    consts = [jnp.asarray(c) for c in
              (_SX, _SY, _S16, _WF, _HF, _WI, _STARTI, _HCOL)]
    rep = lambda i: (0, 0)
    return pl.pallas_call(
        _sampler_body,
        grid=(grid,),
        in_specs=[
            pl.BlockSpec((r, D_MODEL), lambda i: (i, 0)),
            pl.BlockSpec((r, 8), lambda i: (i, 0)),
            pl.BlockSpec((D_MODEL, 256), rep),
            pl.BlockSpec((D_MODEL, 256), rep),
            pl.BlockSpec((1, 128), rep),
            pl.BlockSpec((1, 128), rep),
            pl.BlockSpec((D_MODEL, 128), rep),
            pl.BlockSpec((1, 128), rep),
            pl.BlockSpec((8, 128), rep),
            pl.BlockSpec((8, 128), rep),
            pl.BlockSpec((128, 128), rep),
            pl.BlockSpec((1, 128), rep),
            pl.BlockSpec((1, 128), rep),
            pl.BlockSpec((1, 128), rep),
            pl.BlockSpec((1, 128), rep),
            pl.BlockSpec((1, 128), rep),
        ],
        out_specs=pl.BlockSpec((r, 2, 4, 128), lambda i: (i, 0, 0, 0)),
        out_shape=jax.ShapeDtypeStruct((n, 2, 4, 128), jnp.int32),
    )(qflat, rp8, wh, wl, bx, by, wa, ba, *consts)


def _make_sc_sample(items):
    info = plsc.get_sparse_core_info()
    nw = info.num_cores * info.num_subcores
    hi = items - (items // nw) * nw
    lo_per = items // nw
    mesh = plsc.VectorSubcoreMesh(core_axis_name="c", subcore_axis_name="s")

    @functools.partial(
        pl.kernel,
        mesh=mesh,
        compiler_params=pltpu.CompilerParams(use_tc_tiling_on_sc=False,
                                             needs_layout_passes=False),
        out_type=jax.ShapeDtypeStruct((items, D_MODEL), jnp.float32),
        scratch_types=[
            pltpu.VMEM((5, 2, 4, 128), jnp.int32),
            pltpu.VMEM((4, 4, 128, HEAD_DIM), jnp.bfloat16),
            pltpu.VMEM((2, D_MODEL), jnp.float32),
            pltpu.SemaphoreType.DMA((5,)),
            pltpu.SemaphoreType.DMA((4,)),
            pltpu.SemaphoreType.DMA((2,)),
        ],
    )
    def sc_sample(table_hbm, iw_hbm, out_hbm,
                  iw_v, rows_v, out_v, sem_iw, sem_g, sem_o):
        wid = lax.axis_index("s") * info.num_cores + lax.axis_index("c")
        per = jnp.where(wid < hi, lo_per + 1, lo_per)
        base = jnp.where(wid < hi, wid * (lo_per + 1),
                         wid * lo_per + hi)

        def fire_gathers(slot, rslot):
            for c in range(4):
                pltpu.async_copy(table_hbm.at[iw_v.at[slot, 0, c]],
                                 rows_v.at[rslot, c], sem_g.at[rslot])

        def drain_gathers(rslot):
            for c in range(4):
                pltpu.make_async_copy(table_hbm.at[pl.ds(0, 128)],
                                      rows_v.at[rslot, c],
                                      sem_g.at[rslot]).wait()

        def copy_iw(item, slot):
            pltpu.async_copy(iw_hbm.at[item], iw_v.at[slot], sem_iw.at[slot])

        def drain_iw(slot):
            pltpu.make_async_copy(iw_hbm.at[0], iw_v.at[slot],
                                  sem_iw.at[slot]).wait()

        pltpu.sync_copy(iw_hbm.at[base], iw_v.at[0])
        fire_gathers(0, 0)
        copy_iw(base + 1, 1)
        copy_iw(base + 2, 2)
        copy_iw(base + 3, 3)
        drain_iw(1)
        fire_gathers(1, 1)
        drain_iw(2)
        fire_gathers(2, 2)

        def body(i, carry):
            par = lax.rem(i, 2)
            rslot = lax.rem(i, 4)
            slot = lax.rem(i, 5)

            @pl.when(i + 3 < per)
            def _():
                drain_iw(lax.rem(i + 3, 5))
                fire_gathers(lax.rem(i + 3, 5), lax.rem(i + 3, 4))

            drain_gathers(rslot)

            @pl.when(i + 4 < per)
            def _():
                copy_iw(base + i + 4, lax.rem(i + 4, 5))

            @pl.when(i >= 2)
            def _():
                pltpu.make_async_copy(out_hbm.at[0], out_v.at[par],
                                      sem_o.at[par]).wait()

            for h in range(N_HEADS):
                acc0 = jnp.zeros((16,), jnp.float32)
                acc1 = jnp.zeros((16,), jnp.float32)
                for c in range(4):
                    wv = iw_v[slot, 1, c, pl.ds(h * 16, 16)].astype(
                        jnp.float32)
                    for lp in range(16):
                        j = h * 16 + lp
                        wsc = wv[lp]
                        rev, rod = plsc.unpack(rows_v[rslot, c, j, :],
                                               format=plsc.PackFormat.INTERLEAVED)
                        acc0 = acc0 + wsc * rev
                        acc1 = acc1 + wsc * rod
                out_v[par, pl.ds(h * HEAD_DIM, 16)] = acc0
                out_v[par, pl.ds(h * HEAD_DIM + 16, 16)] = acc1

            pltpu.async_copy(out_v.at[par], out_hbm.at[base + i],
                             sem_o.at[par])
            return carry

        lax.fori_loop(0, per, body, 0)
        for par in range(2):
            pltpu.make_async_copy(out_hbm.at[0], out_v.at[par],
                                  sem_o.at[par]).wait()

    return sc_sample


def kernel(query, reference_points, value, spatial_shapes, W_off, b_off,
           W_attn, b_attn, W_val, b_val, W_out, b_out):
    del spatial_shapes
    bsz, nq, _ = query.shape

    table = _matmul(value.reshape(bsz * NV, D_MODEL), W_val, b_val, 2720,
                    out_dtype=jnp.bfloat16)
    table = table.reshape(bsz * NV * N_HEADS, HEAD_DIM)

    qflat = query.reshape(bsz * nq, D_MODEL)
    rp8 = reference_points.reshape(bsz * nq, 2 * N_LEVELS)
    wxy = jnp.concatenate([W_off[:, 0::2], W_off[:, 1::2]], axis=1)
    wh = wxy.astype(jnp.bfloat16)
    wl = (wxy - wh.astype(jnp.float32)).astype(jnp.bfloat16)
    bx = b_off[0::2].reshape(1, 128)
    by = b_off[1::2].reshape(1, 128)
    iw = _sampler(qflat, rp8, wh, wl, bx, by,
                  W_attn, b_attn.reshape(1, 128))

    sampled = _make_sc_sample(bsz * nq)(table, iw)

    out = _matmul(sampled, W_out[_UNPACK_PERM] * (2.0 ** -20), b_out, 1200)
    return out.reshape(bsz, nq, D_MODEL)


# speedup vs baseline: 1.0066x; 1.0066x over previous
import functools

import jax
import jax.numpy as jnp
import numpy as np
from jax import lax
from jax.experimental import pallas as pl
from jax.experimental.pallas import tpu as pltpu
from jax.experimental.pallas import tpu_sc as plsc

D_MODEL = 256
N_HEADS = 8
N_LEVELS = 4
N_POINTS = 4
HEAD_DIM = 32
SPATIAL_SHAPES = [(64, 64), (32, 32), (16, 16), (8, 8)]
NV = sum(h * w for h, w in SPATIAL_SHAPES)

_LVL = (np.arange(128) // 4) % 4
_WF = np.array([SPATIAL_SHAPES[l][1] for l in _LVL], np.float32).reshape(1, 128)
_HF = np.array([SPATIAL_SHAPES[l][0] for l in _LVL], np.float32).reshape(1, 128)
_WI = _WF.astype(np.int32)
_STARTS = np.cumsum([0] + [h * w for h, w in SPATIAL_SHAPES])[:4]
_STARTI = np.array([_STARTS[l] for l in _LVL], np.int32).reshape(1, 128)
_HCOL = (np.arange(128) // 16).astype(np.int32).reshape(1, 128)
_SX = np.zeros((8, 128), np.float32)
_SY = np.zeros((8, 128), np.float32)
for _l in range(4):
    _SX[2 * _l] = (_LVL == _l)
    _SY[2 * _l + 1] = (_LVL == _l)
_S16 = ((np.arange(128)[:, None] // 16) == (np.arange(128)[None, :] // 16)
        ).astype(np.float32)
_UNPACK_PERM = np.concatenate(
    [h * 32 + np.concatenate([np.arange(16) * 2, np.arange(16) * 2 + 1])
     for h in range(N_HEADS)])


def _matmul_body(out_dtype, x_ref, w_ref, b_ref, o_ref):
    o_ref[...] = (
        jnp.dot(x_ref[...], w_ref[...], preferred_element_type=jnp.float32)
        + b_ref[...]
    ).astype(out_dtype)


def _matmul(x, w, b, block_rows, out_dtype=jnp.float32):
    n, k = x.shape
    m = w.shape[1]
    grid = n // block_rows
    return pl.pallas_call(
        functools.partial(_matmul_body, out_dtype),
        grid=(grid,),
        in_specs=[
            pl.BlockSpec((block_rows, k), lambda i: (i, 0)),
            pl.BlockSpec((k, m), lambda i: (0, 0)),
            pl.BlockSpec((1, m), lambda i: (0, 0)),
        ],
        out_specs=pl.BlockSpec((block_rows, m), lambda i: (i, 0)),
        out_shape=jax.ShapeDtypeStruct((n, m), out_dtype),
    )(x, w, b.reshape(1, m))


_ROWS_PER_STEP = 3600


def _sampler_body(q_ref, rp_ref, wh_ref, wl_ref, bx_ref, by_ref,
                  wa_ref, ba_ref, sx_ref, sy_ref, s16_ref, wf_ref, hf_ref,
                  wi_ref, starti_ref, hcol_ref, iw_ref):
    step = pl.program_id(0)
    r = _ROWS_PER_STEP
    q = q_ref[...]
    qh = q.astype(jnp.bfloat16)
    ql = (q - qh.astype(jnp.float32)).astype(jnp.bfloat16)
    wh = wh_ref[...]
    oxy = (jnp.dot(qh, wh, preferred_element_type=jnp.float32)
           + jnp.dot(qh, wl_ref[...], preferred_element_type=jnp.float32)
           + jnp.dot(ql, wh, preferred_element_type=jnp.float32))
    offx = oxy[:, 0:128] + bx_ref[...]
    offy = oxy[:, 128:256] + by_ref[...]
    a = jnp.dot(q, wa_ref[...], preferred_element_type=jnp.float32) + ba_ref[...]
    e = jnp.exp(a - jnp.max(a, axis=1, keepdims=True))
    attn = e / jnp.dot(e, s16_ref[...], preferred_element_type=jnp.float32)
    rpf = rp_ref[...]
    rph = rpf.astype(jnp.bfloat16)
    rpl = (rpf - rph.astype(jnp.float32)).astype(jnp.bfloat16)
    sx = sx_ref[...].astype(jnp.bfloat16)
    sy = sy_ref[...].astype(jnp.bfloat16)
    rx = (jnp.dot(rph, sx, preferred_element_type=jnp.float32)
          + jnp.dot(rpl, sx, preferred_element_type=jnp.float32))
    ry = (jnp.dot(rph, sy, preferred_element_type=jnp.float32)
          + jnp.dot(rpl, sy, preferred_element_type=jnp.float32))
    wf = wf_ref[...]
    hf = hf_ref[...]
    x = rx * wf + offx - 0.5
    y = ry * hf + offy - 0.5
    x0 = jnp.floor(x)
    fx = x - x0
    y0 = jnp.floor(y)
    fy = y - y0
    rg = step * r + lax.broadcasted_iota(jnp.int32, (r, 128), 0)
    bvec = ((rg >= 900).astype(jnp.int32) + (rg >= 1800).astype(jnp.int32)
            + (rg >= 2700).astype(jnp.int32))
    base = bvec * (NV * N_HEADS)
    for cy in (0, 1):
        yf = y0 + cy
        vy = (yf >= 0.0) & (yf <= hf - 1.0)
        yc = jnp.clip(yf, 0.0, hf - 1.0).astype(jnp.int32)
        wyv = fy if cy else 1.0 - fy
        for cx in (0, 1):
            xf = x0 + cx
            vx = (xf >= 0.0) & (xf <= wf - 1.0)
            xc = jnp.clip(xf, 0.0, wf - 1.0).astype(jnp.int32)
            wxv = fx if cx else 1.0 - fx
            wgt = jnp.where(vx & vy, attn * wxv * wyv, 0.0)
            pix = starti_ref[...] + yc * wi_ref[...] + xc
            idx = base + pix * N_HEADS + hcol_ref[...]
            c = cy * 2 + cx
            iw_ref[:, 0, c, :] = idx
            iw_ref[:, 1, c, :] = (wgt * (2.0 ** 20)).astype(jnp.int32)


def _sampler(qflat, rp8, wh, wl, bx, by, wa, ba):
    n = qflat.shape[0]
    grid = n // _ROWS_PER_STEP
    r = _ROWS_PER_STEP
    consts = [jnp.asarray(c) for c in
              (_SX, _SY, _S16, _WF, _HF, _WI, _STARTI, _HCOL)]
    rep = lambda i: (0, 0)
    return pl.pallas_call(
        _sampler_body,
        grid=(grid,),
        in_specs=[
            pl.BlockSpec((r, D_MODEL), lambda i: (i, 0)),
            pl.BlockSpec((r, 8), lambda i: (i, 0)),
            pl.BlockSpec((D_MODEL, 256), rep),
            pl.BlockSpec((D_MODEL, 256), rep),
            pl.BlockSpec((1, 128), rep),
            pl.BlockSpec((1, 128), rep),
            pl.BlockSpec((D_MODEL, 128), rep),
            pl.BlockSpec((1, 128), rep),
            pl.BlockSpec((8, 128), rep),
            pl.BlockSpec((8, 128), rep),
            pl.BlockSpec((128, 128), rep),
            pl.BlockSpec((1, 128), rep),
            pl.BlockSpec((1, 128), rep),
            pl.BlockSpec((1, 128), rep),
            pl.BlockSpec((1, 128), rep),
            pl.BlockSpec((1, 128), rep),
        ],
        out_specs=pl.BlockSpec((r, 2, 4, 128), lambda i: (i, 0, 0, 0)),
        out_shape=jax.ShapeDtypeStruct((n, 2, 4, 128), jnp.int32),
    )(qflat, rp8, wh, wl, bx, by, wa, ba, *consts)


def _make_sc_sample(items):
    info = plsc.get_sparse_core_info()
    nw = info.num_cores * info.num_subcores
    hi = items - (items // nw) * nw
    lo_per = items // nw
    mesh = plsc.VectorSubcoreMesh(core_axis_name="c", subcore_axis_name="s")

    @functools.partial(
        pl.kernel,
        mesh=mesh,
        compiler_params=pltpu.CompilerParams(use_tc_tiling_on_sc=False,
                                             needs_layout_passes=False),
        out_type=jax.ShapeDtypeStruct((items, D_MODEL), jnp.float32),
        scratch_types=[
            pltpu.VMEM((5, 2, 4, 128), jnp.int32),
            pltpu.VMEM((4, 4, 128, HEAD_DIM), jnp.bfloat16),
            pltpu.VMEM((2, D_MODEL), jnp.float32),
            pltpu.SemaphoreType.DMA((5,)),
            pltpu.SemaphoreType.DMA((4,)),
            pltpu.SemaphoreType.DMA((2,)),
        ],
    )
    def sc_sample(table_hbm, iw_hbm, out_hbm,
                  iw_v, rows_v, out_v, sem_iw, sem_g, sem_o):
        wid = lax.axis_index("s") * info.num_cores + lax.axis_index("c")
        per = jnp.where(wid < hi, lo_per + 1, lo_per)
        base = jnp.where(wid < hi, wid * (lo_per + 1),
                         wid * lo_per + hi)

        def fire_gathers(slot, rslot):
            for c in range(4):
                pltpu.async_copy(table_hbm.at[iw_v.at[slot, 0, c]],
                                 rows_v.at[rslot, c], sem_g.at[rslot])

        def drain_gathers(rslot):
            for c in range(4):
                pltpu.make_async_copy(table_hbm.at[pl.ds(0, 128)],
                                      rows_v.at[rslot, c],
                                      sem_g.at[rslot]).wait()

        def copy_iw(item, slot):
            pltpu.async_copy(iw_hbm.at[item], iw_v.at[slot], sem_iw.at[slot])

        def drain_iw(slot):
            pltpu.make_async_copy(iw_hbm.at[0], iw_v.at[slot],
                                  sem_iw.at[slot]).wait()

        pltpu.sync_copy(iw_hbm.at[base], iw_v.at[0])
        fire_gathers(0, 0)
        copy_iw(base + 1, 1)
        copy_iw(base + 2, 2)
        copy_iw(base + 3, 3)
        drain_iw(1)
        fire_gathers(1, 1)
        drain_iw(2)
        fire_gathers(2, 2)

        def body(i, carry):
            par = lax.rem(i, 2)
            rslot = lax.rem(i, 4)
            slot = lax.rem(i, 5)

            @pl.when(i + 3 < per)
            def _():
                drain_iw(lax.rem(i + 3, 5))
                fire_gathers(lax.rem(i + 3, 5), lax.rem(i + 3, 4))

            drain_gathers(rslot)

            @pl.when(i + 4 < per)
            def _():
                copy_iw(base + i + 4, lax.rem(i + 4, 5))

            @pl.when(i >= 2)
            def _():
                pltpu.make_async_copy(out_hbm.at[0], out_v.at[par],
                                      sem_o.at[par]).wait()

            for h in range(N_HEADS):
                acc0 = jnp.zeros((16,), jnp.float32)
                acc1 = jnp.zeros((16,), jnp.float32)
                for c in range(4):
                    wv = iw_v[slot, 1, c, pl.ds(h * 16, 16)].astype(
                        jnp.float32)
                    for lp in range(16):
                        j = h * 16 + lp
                        wsc = wv[lp]
                        rev, rod = plsc.unpack(rows_v[rslot, c, j, :],
                                               format=plsc.PackFormat.INTERLEAVED)
                        acc0 = acc0 + wsc * rev
                        acc1 = acc1 + wsc * rod
                out_v[par, pl.ds(h * HEAD_DIM, 16)] = acc0
                out_v[par, pl.ds(h * HEAD_DIM + 16, 16)] = acc1

            pltpu.async_copy(out_v.at[par], out_hbm.at[base + i],
                             sem_o.at[par])
            return carry

        lax.fori_loop(0, per, body, 0)
        for par in range(2):
            pltpu.make_async_copy(out_hbm.at[0], out_v.at[par],
                                  sem_o.at[par]).wait()

    return sc_sample


def kernel(query, reference_points, value, spatial_shapes, W_off, b_off,
           W_attn, b_attn, W_val, b_val, W_out, b_out):
    del spatial_shapes
    bsz, nq, _ = query.shape

    table = _matmul(value.reshape(bsz * NV, D_MODEL), W_val, b_val, 5440,
                    out_dtype=jnp.bfloat16)
    table = table.reshape(bsz * NV * N_HEADS, HEAD_DIM)

    qflat = query.reshape(bsz * nq, D_MODEL)
    rp8 = reference_points.reshape(bsz * nq, 2 * N_LEVELS)
    wxy = jnp.concatenate([W_off[:, 0::2], W_off[:, 1::2]], axis=1)
    wh = wxy.astype(jnp.bfloat16)
    wl = (wxy - wh.astype(jnp.float32)).astype(jnp.bfloat16)
    bx = b_off[0::2].reshape(1, 128)
    by = b_off[1::2].reshape(1, 128)
    iw = _sampler(qflat, rp8, wh, wl, bx, by,
                  W_attn, b_attn.reshape(1, 128))

    sampled = _make_sc_sample(bsz * nq)(table, iw)

    out = _matmul(sampled, W_out[_UNPACK_PERM] * (2.0 ** -20), b_out, 1200)
    return out.reshape(bsz, nq, D_MODEL)
